# Initial kernel scaffold; baseline (speedup 1.0000x reference)
import jax
import jax.numpy as jnp
from jax.experimental import pallas as pl


def kernel(A_adj_idx, A_adj_val, A_vp_row, A_vp_col, A_vp_val, A_vc_row, A_vc_col, A_vc_val, B_adj_idx, B_adj_val, B_vp_row, B_vp_col, B_vp_val, B_vc_row, B_vc_col, B_vc_val, A_emb, A_price_emb, A_cat_emb, B_emb, B_price_emb, B_cat_emb, W_pv, b_pv, W_pc, b_pc, W_v1, b_v1):
    raise NotImplementedError("write your pallas kernel here")


# speedup vs baseline: 1.0606x; 1.0606x over previous
import functools

import jax
import jax.numpy as jnp
from jax import lax
from jax.experimental import pallas as pl
from jax.experimental.pallas import tpu as pltpu
from jax.experimental.pallas import tpu_sc as plsc

N_ITEMS = 50000
N_EDGES = 800000
N_PRICE = 100
N_CAT = 200
EMB = 64
HALF_D = 32

NCORES = 2
NSUB = 16

CK = 64
EPT = 50176
NE_PAD = EPT * NSUB
SEG_CHUNKS = EPT // CK
GROWS = N_ITEMS // 2
AROWS_REAL = N_ITEMS // 4
APT = 784
AROWS = NSUB * APT
AZ_FULL = APT // CK
AZ_TAIL = APT - AZ_FULL * CK

NV = N_ITEMS
VPT = 3200
NV_PAD = VPT * NSUB
VCHUNKS = VPT // CK
REAL_WORDS = 1250000
SP = 1251328
STRIPE = SP // NSUB
CW = 8192
ZFULL = STRIPE // CW
ZTAIL = STRIPE - ZFULL * CW

_mesh = plsc.VectorSubcoreMesh(
    core_axis_name="c", subcore_axis_name="s", num_cores=NCORES,
    num_subcores=NSUB)


def _seg_body(dst_hbm, src_hbm, val_hbm, table_hbm, out_hbm,
              acc_sp, stage, gbuf, dst_v, src_v, val_v, gidx_v, goff_v,
              sidx_v, soff_v, sem):
    c = lax.axis_index("c")
    s = lax.axis_index("s")
    z16 = jnp.zeros((16,), jnp.float32)

    def zfill(j, carry):
        for q in range(8):
            stage[j, pl.ds(q * 16, 16)] = z16
        return carry

    lax.fori_loop(0, CK, zfill, 0)

    def zcopy(k, carry):
        pltpu.sync_copy(stage, acc_sp.at[pl.ds(s * APT + k * CK, CK)])
        return carry

    lax.fori_loop(0, AZ_FULL, zcopy, 0)
    pltpu.sync_copy(stage.at[pl.ds(0, AZ_TAIL)],
                    acc_sp.at[pl.ds(s * APT + AZ_FULL * CK, AZ_TAIL)])
    plsc.subcore_barrier()

    base_e = s * EPT
    coff = c * HALF_D

    def chunk(g, carry):
        eb = base_e + g * CK
        pltpu.sync_copy(dst_hbm.at[pl.ds(eb, CK)], dst_v)
        pltpu.sync_copy(src_hbm.at[pl.ds(eb, CK)], src_v)
        pltpu.sync_copy(val_hbm.at[pl.ds(eb, CK)], val_v)
        for j in range(CK // 16):
            sl = pl.ds(j * 16, 16)
            sv = src_v[sl]
            dv = dst_v[sl]
            gidx_v[sl] = sv >> 1
            goff_v[sl] = (sv & 1) * 64 + coff
            sidx_v[sl] = dv >> 2
            soff_v[sl] = (dv & 3) * 32
        pltpu.async_copy(table_hbm.at[gidx_v], gbuf, sem).wait()

        for gg in range(CK // 16):
            osl = pl.ds(gg * 16, 16)
            vv = val_v[osl]
            gofs = goff_v[osl]
            sofs = soff_v[osl]
            for l in range(16):
                j = gg * 16 + l
                vbc = jnp.full((16,), vv[l], jnp.float32)
                go = gofs[l]
                so = sofs[l]
                r0 = gbuf[j, pl.ds(go, 16)] * vbc
                r1 = gbuf[j, pl.ds(go + 16, 16)] * vbc
                for q in range(4):
                    stage[j, pl.ds(q * 32, 16)] = z16
                    stage[j, pl.ds(q * 32 + 16, 16)] = z16
                stage[j, pl.ds(so, 16)] = r0
                stage[j, pl.ds(so + 16, 16)] = r1
        pltpu.sync_copy(stage, acc_sp.at[sidx_v], add=True)
        return carry

    lax.fori_loop(0, SEG_CHUNKS, chunk, 0)
    plsc.subcore_barrier()

    def wcopy(k, carry):
        pltpu.sync_copy(acc_sp.at[pl.ds(s * APT + k * CK, CK)], stage)
        pltpu.sync_copy(
            stage, out_hbm.at[pl.ds((c * NSUB + s) * APT + k * CK, CK)])
        return carry

    lax.fori_loop(0, AZ_FULL, wcopy, 0)
    toff = s * APT + AZ_FULL * CK
    pltpu.sync_copy(acc_sp.at[pl.ds(toff, AZ_TAIL)],
                    stage.at[pl.ds(0, AZ_TAIL)])
    pltpu.sync_copy(
        stage.at[pl.ds(0, AZ_TAIL)],
        out_hbm.at[pl.ds((c * NSUB + s) * APT + AZ_FULL * CK, AZ_TAIL)])


_seg_call = pl.kernel(
    _seg_body,
    out_type=jax.ShapeDtypeStruct((2 * AROWS, 128), jnp.float32),
    mesh=_mesh,
    scratch_types=[
        pltpu.VMEM_SHARED((AROWS, 128), jnp.float32),
        pltpu.VMEM((CK, 128), jnp.float32),
        pltpu.VMEM((CK, 128), jnp.float32),
        pltpu.VMEM((CK,), jnp.int32),
        pltpu.VMEM((CK,), jnp.int32),
        pltpu.VMEM((CK,), jnp.float32),
        pltpu.VMEM((CK,), jnp.int32),
        pltpu.VMEM((CK,), jnp.int32),
        pltpu.VMEM((CK,), jnp.int32),
        pltpu.VMEM((CK,), jnp.int32),
        pltpu.SemaphoreType.DMA,
    ],
)


def _mask_body(n2, ch_rows, passes, row_hbm, col_hbm, val_hbm,
               out_hbm, acc_sp, zeros_v, bounce_v, r_v, c_v, v_v, idx_v,
               vm_v):
    c = lax.axis_index("c")
    s = lax.axis_index("s")
    dump = jnp.full((16,), REAL_WORDS, jnp.int32)
    n2v = jnp.full((16,), n2, jnp.int32)

    def zfill(j, carry):
        zeros_v[pl.ds(j * 16, 16)] = jnp.zeros((16,), jnp.float32)
        return carry

    lax.fori_loop(0, CW // 16, zfill, 0)

    for p in range(passes):
        base_r = c * (N_ITEMS // 2) + p * ch_rows
        lo = jnp.full((16,), 1, jnp.int32) * base_r
        hi = lo + ch_rows
        def zcopy(k, carry):
            pltpu.sync_copy(zeros_v,
                            acc_sp.at[pl.ds(s * STRIPE + k * CW, CW)])
            return carry

        lax.fori_loop(0, ZFULL, zcopy, 0)
        pltpu.sync_copy(
            zeros_v.at[pl.ds(0, ZTAIL)],
            acc_sp.at[pl.ds(s * STRIPE + ZFULL * CW, ZTAIL)])
        plsc.subcore_barrier()

        def chunk(g, carry):
            eb = s * VPT + g * CK
            pltpu.sync_copy(row_hbm.at[pl.ds(eb, CK)], r_v)
            pltpu.sync_copy(col_hbm.at[pl.ds(eb, CK)], c_v)
            pltpu.sync_copy(val_hbm.at[pl.ds(eb, CK)], v_v)
            for j in range(CK // 16):
                sl = pl.ds(j * 16, 16)
                r16 = r_v[sl]
                m = (r16 >= lo) & (r16 < hi)
                idx_v[sl] = jnp.where(m, (r16 - lo) * n2v + c_v[sl], dump)
                vm_v[sl] = jnp.where(m, v_v[sl], jnp.zeros((16,), jnp.float32))
            pltpu.sync_copy(vm_v, acc_sp.at[idx_v], add=True)
            return carry

        lax.fori_loop(0, VCHUNKS, chunk, 0)
        plsc.subcore_barrier()

        out_base = (c * passes + p) * SP

        def wcopy(k, carry):
            off = s * STRIPE + k * CW
            pltpu.sync_copy(acc_sp.at[pl.ds(off, CW)], bounce_v)
            pltpu.sync_copy(bounce_v, out_hbm.at[pl.ds(out_base + off, CW)])
            return carry

        lax.fori_loop(0, ZFULL, wcopy, 0)
        toff = s * STRIPE + ZFULL * CW
        pltpu.sync_copy(acc_sp.at[pl.ds(toff, ZTAIL)],
                        bounce_v.at[pl.ds(0, ZTAIL)])
        pltpu.sync_copy(bounce_v.at[pl.ds(0, ZTAIL)],
                        out_hbm.at[pl.ds(out_base + toff, ZTAIL)])


def _make_mask_call(n2, ch_rows, passes):
    return pl.kernel(
        functools.partial(_mask_body, n2, ch_rows, passes),
        out_type=jax.ShapeDtypeStruct((2 * passes * SP,), jnp.float32),
        mesh=_mesh,
        scratch_types=[
            pltpu.VMEM_SHARED((SP,), jnp.float32),
            pltpu.VMEM((CW,), jnp.float32),
            pltpu.VMEM((CW,), jnp.float32),
            pltpu.VMEM((CK,), jnp.int32),
            pltpu.VMEM((CK,), jnp.int32),
            pltpu.VMEM((CK,), jnp.float32),
            pltpu.VMEM((CK,), jnp.int32),
            pltpu.VMEM((CK,), jnp.float32),
        ],
    )


_mask_call_p = _make_mask_call(N_PRICE, 12500, 2)
_mask_call_c = _make_mask_call(N_CAT, 6250, 4)


BLK = 512
GRID = (N_ITEMS + BLK - 1) // BLK


def _tc_body(item_ref, dp_ref, dc_ref, pemb_ref, cemb_ref, wpv_ref, wpc_ref,
             wv1_ref, bv1_ref, out_ref):
    it = item_ref[...]

    def gate(d_ref, emb2_ref, w_ref):
        emb2 = emb2_ref[...]
        itw = lax.dot_general(it, w_ref[...], (((1,), (0,)), ((), ())),
                              preferred_element_type=jnp.float32)
        alpha = lax.dot_general(itw, emb2, (((1,), (1,)), ((), ())),
                                preferred_element_type=jnp.float32)
        m = jnp.max(alpha, axis=1, keepdims=True)
        e = jnp.exp(alpha - m)
        sm = e / jnp.sum(e, axis=1, keepdims=True)
        w = sm * d_ref[...]
        s = jnp.sum(w, axis=1, keepdims=True) + 1e-8
        return lax.dot_general(w / s, emb2, (((1,), (0,)), ((), ())),
                               preferred_element_type=jnp.float32)

    p_info = gate(dp_ref, pemb_ref, wpv_ref)
    c_info = gate(dc_ref, cemb_ref, wpc_ref)
    wv1 = wv1_ref[...]
    acc = lax.dot_general(it, wv1[:, 0:EMB], (((1,), (1,)), ((), ())),
                          preferred_element_type=jnp.float32)
    acc += lax.dot_general(p_info, wv1[:, EMB:2 * EMB],
                           (((1,), (1,)), ((), ())),
                           preferred_element_type=jnp.float32)
    acc += lax.dot_general(c_info, wv1[:, 2 * EMB:3 * EMB],
                           (((1,), (1,)), ((), ())),
                           preferred_element_type=jnp.float32)
    out_ref[...] = jnp.maximum(acc + bv1_ref[...], 0.0)


_tc_call = pl.pallas_call(
    _tc_body,
    out_shape=jax.ShapeDtypeStruct((N_ITEMS, EMB), jnp.float32),
    grid=GRID,
    in_specs=[
        pl.BlockSpec((BLK, EMB), lambda i: (i, 0)),
        pl.BlockSpec((BLK, N_PRICE), lambda i: (i, 0)),
        pl.BlockSpec((BLK, N_CAT), lambda i: (i, 0)),
        pl.BlockSpec((N_PRICE, EMB), lambda i: (0, 0)),
        pl.BlockSpec((N_CAT, EMB), lambda i: (0, 0)),
        pl.BlockSpec((EMB, EMB), lambda i: (0, 0)),
        pl.BlockSpec((EMB, EMB), lambda i: (0, 0)),
        pl.BlockSpec((EMB, 3 * EMB), lambda i: (0, 0)),
        pl.BlockSpec((1, EMB), lambda i: (0, 0)),
    ],
    out_specs=pl.BlockSpec((BLK, EMB), lambda i: (i, 0)),
)


def _pad_edges(adj_idx, adj_val):
    pad = NE_PAD - N_EDGES
    dst = jnp.concatenate([adj_idx[0], jnp.zeros((pad,), jnp.int32)])
    src = jnp.concatenate([adj_idx[1], jnp.zeros((pad,), jnp.int32)])
    val = jnp.concatenate([adj_val, jnp.zeros((pad,), jnp.float32)])
    return dst, src, val


def _pad_entries(row, col, val):
    pad = NV_PAD - NV
    row = jnp.concatenate([row, jnp.full((pad,), N_ITEMS, jnp.int32)])
    col = jnp.concatenate([col, jnp.zeros((pad,), jnp.int32)])
    val = jnp.concatenate([val, jnp.zeros((pad,), jnp.float32)])
    return row, col, val


def _domain(adj_idx, adj_val, vp_row, vp_col, vp_val, vc_row, vc_col, vc_val,
            emb, p_emb, c_emb, W_pv, W_pc, W_v1, b_v1):
    dst, src, val = _pad_edges(adj_idx, adj_val)
    table = emb.reshape(GROWS, 128)
    raw = _seg_call(dst, src, val, table)
    halves = raw.reshape(2, AROWS, 4, HALF_D)[:, :AROWS_REAL]
    halves = halves.reshape(2, N_ITEMS, HALF_D)
    item = jnp.concatenate([halves[0], halves[1]], axis=1)

    pr, pc, pv = _pad_entries(vp_row, vp_col, vp_val)
    dp = _mask_call_p(pr, pc, pv).reshape(2 * 2, SP)[:, :REAL_WORDS]
    dp = dp.reshape(N_ITEMS, N_PRICE)
    cr, cc, cv = _pad_entries(vc_row, vc_col, vc_val)
    dc = _mask_call_c(cr, cc, cv).reshape(2 * 4, SP)[:, :REAL_WORDS]
    dc = dc.reshape(N_ITEMS, N_CAT)

    return _tc_call(item, dp, dc, p_emb, c_emb, W_pv, W_pc, W_v1,
                    b_v1.reshape(1, EMB))


def kernel(A_adj_idx, A_adj_val, A_vp_row, A_vp_col, A_vp_val, A_vc_row,
           A_vc_col, A_vc_val, B_adj_idx, B_adj_val, B_vp_row, B_vp_col,
           B_vp_val, B_vc_row, B_vc_col, B_vc_val, A_emb, A_price_emb,
           A_cat_emb, B_emb, B_price_emb, B_cat_emb, W_pv, b_pv, W_pc, b_pc,
           W_v1, b_v1):
    out_a = _domain(A_adj_idx, A_adj_val, A_vp_row, A_vp_col, A_vp_val,
                    A_vc_row, A_vc_col, A_vc_val, A_emb, A_price_emb,
                    A_cat_emb, W_pv, W_pc, W_v1, b_v1)
    out_b = _domain(B_adj_idx, B_adj_val, B_vp_row, B_vp_col, B_vp_val,
                    B_vc_row, B_vc_col, B_vc_val, B_emb, B_price_emb,
                    B_cat_emb, W_pv, W_pc, W_v1, b_v1)
    return jnp.concatenate([out_a, out_b], axis=0)
